# baseline (speedup 1.0000x reference)
import math

import jax
import jax.numpy as jnp
from jax.experimental import pallas as pl
from jax.experimental.pallas import tpu as pltpu

BSZ = 8
C_LEN = 512
QN = 64
QL = 64
D = 512
H = 512


def _question(cp_scr, w2_ref, qb, avg_b):
    cat = jnp.concatenate([qb, avg_b], axis=1)
    qp = jnp.dot(cat, w2_ref[...], preferred_element_type=jnp.float32)
    qp_b = jax.nn.relu(qp).astype(jnp.bfloat16)
    s = jax.lax.dot_general(
        cp_scr[...], qp_b, (((1,), (1,)), ((), ())),
        preferred_element_type=jnp.float32,
    ) * (1.0 / math.sqrt(H))
    s = s - jnp.max(s, axis=1, keepdims=True)
    e = jnp.exp(s)
    attn = (e / jnp.sum(e, axis=1, keepdims=True)).astype(jnp.bfloat16)
    return jnp.dot(attn, cat, preferred_element_type=jnp.float32)


def _fused_kernel(tags_ref, ctxa_ref, ctxb_ref, q_ref, w1_ref, w2_ref,
                  out_ref, cp_scr, qsum_scr, pos_ref):
    k = pl.program_id(0)
    t0 = tags_ref[2 * k]
    t1 = tags_ref[2 * k + 1]
    tprev = tags_ref[jnp.maximum(2 * k - 1, 0)]
    seg0 = (k == 0) | (t0 != tprev)
    seg1 = t1 != t0

    pos = pos_ref[0]
    pos0 = jnp.where(seg0, 0, pos)
    inv0 = jnp.where(pos0 == 0, 0.0, 1.0 / pos0.astype(jnp.float32))
    pos1 = jnp.where(seg1, 0, pos0 + 1)
    inv1 = jnp.where(pos1 == 0, 0.0, 1.0 / pos1.astype(jnp.float32))

    qf0 = q_ref[0, 0]
    qf1 = q_ref[0, 1]
    qsum = jnp.where(seg0, 0.0, qsum_scr[...])
    avg0 = qsum * inv0
    qe1 = jnp.where(seg1, 0.0, qsum + qf0)
    avg1 = qe1 * inv1
    qsum_scr[...] = qe1 + qf1
    pos_ref[0] = pos1 + 1

    @pl.when(seg0)
    def _():
        cp_scr[...] = jax.nn.relu(
            jnp.dot(ctxa_ref[0], w1_ref[...], preferred_element_type=jnp.float32)
        ).astype(jnp.bfloat16)

    out_ref[0, 0] = _question(cp_scr, w2_ref,
                              qf0.astype(jnp.bfloat16), avg0.astype(jnp.bfloat16))

    @pl.when(seg1)
    def _():
        cp_scr[...] = jax.nn.relu(
            jnp.dot(ctxb_ref[0], w1_ref[...], preferred_element_type=jnp.float32)
        ).astype(jnp.bfloat16)

    out_ref[0, 1] = _question(cp_scr, w2_ref,
                              qf1.astype(jnp.bfloat16), avg1.astype(jnp.bfloat16))


def kernel(contexts, questions, tags, W1, W2):
    tags32 = tags.astype(jnp.int32)
    ctx_b = contexts.astype(jnp.bfloat16)
    w1_b = W1.astype(jnp.bfloat16)
    w2_b = W2.astype(jnp.bfloat16)
    q4 = questions.reshape(QN // 2, 2, QL, D)

    out = pl.pallas_call(
        _fused_kernel,
        grid_spec=pltpu.PrefetchScalarGridSpec(
            num_scalar_prefetch=1,
            grid=(QN // 2,),
            in_specs=[
                pl.BlockSpec((1, C_LEN, D), lambda k, t: (t[2 * k], 0, 0)),
                pl.BlockSpec((1, C_LEN, D), lambda k, t: (t[2 * k + 1], 0, 0)),
                pl.BlockSpec((1, 2, QL, D), lambda k, t: (k, 0, 0, 0)),
                pl.BlockSpec((D, H), lambda k, t: (0, 0)),
                pl.BlockSpec((2 * D, H), lambda k, t: (0, 0)),
            ],
            out_specs=pl.BlockSpec((1, 2, C_LEN, 2 * D), lambda k, t: (k, 0, 0, 0)),
            scratch_shapes=[
                pltpu.VMEM((C_LEN, H), jnp.bfloat16),
                pltpu.VMEM((QL, D), jnp.float32),
                pltpu.SMEM((1,), jnp.int32),
            ],
        ),
        out_shape=jax.ShapeDtypeStruct((QN // 2, 2, C_LEN, 2 * D), jnp.float32),
        compiler_params=pltpu.CompilerParams(dimension_semantics=("arbitrary",)),
    )(tags32, ctx_b, ctx_b, q4, w1_b, w2_b)

    return out.reshape(QN, C_LEN, 2 * D)


# speedup vs baseline: 1.1816x; 1.1816x over previous
import math

import jax
import jax.numpy as jnp
from jax.experimental import pallas as pl
from jax.experimental.pallas import tpu as pltpu

BSZ = 8
C_LEN = 512
QN = 64
QL = 64
D = 512
H = 512
G = 4


def _question(cp, w2_ref, qb, avg_b):
    cat = jnp.concatenate([qb, avg_b], axis=1)
    qp = jnp.dot(cat, w2_ref[...], preferred_element_type=jnp.float32)
    qp_b = jax.nn.relu(qp).astype(jnp.bfloat16)
    s = jax.lax.dot_general(
        cp, qp_b, (((1,), (1,)), ((), ())),
        preferred_element_type=jnp.float32,
    ) * (1.0 / math.sqrt(H))
    s = s - jnp.max(s, axis=1, keepdims=True)
    e = jnp.exp(s)
    attn = (e / jnp.sum(e, axis=1, keepdims=True)).astype(jnp.bfloat16)
    return jnp.dot(attn, cat, preferred_element_type=jnp.float32)


def _fused_kernel(tags_ref, ctx_ref, q_ref, w1_ref, w2_ref,
                  out_ref, cp_all, qsum_scr, pos_ref):
    k = pl.program_id(0)

    @pl.when(k == 0)
    def _():
        for b in range(BSZ):
            cp_all[b] = jax.nn.relu(
                jnp.dot(ctx_ref[b], w1_ref[...], preferred_element_type=jnp.float32)
            ).astype(jnp.bfloat16)

    p = pos_ref[0]
    S = qsum_scr[...]
    tprev = tags_ref[jnp.maximum(G * k - 1, 0)]
    for i in range(G):
        ti = tags_ref[G * k + i]
        seg = (ti != tprev) if i else ((k == 0) | (ti != tprev))
        p = jnp.where(seg, 0, p)
        S = jnp.where(seg, 0.0, S)
        inv = jnp.where(p == 0, 0.0, 1.0 / p.astype(jnp.float32))
        avg_b = (S * inv).astype(jnp.bfloat16)
        qf = q_ref[0, i]
        out_ref[0, i] = _question(cp_all[ti], w2_ref,
                                  qf.astype(jnp.bfloat16), avg_b)
        S = S + qf
        p = p + 1
        tprev = ti
    qsum_scr[...] = S
    pos_ref[0] = p


def kernel(contexts, questions, tags, W1, W2):
    tags32 = tags.astype(jnp.int32)
    ctx_b = contexts.astype(jnp.bfloat16)
    w1_b = W1.astype(jnp.bfloat16)
    w2_b = W2.astype(jnp.bfloat16)
    q4 = questions.reshape(QN // G, G, QL, D)

    out = pl.pallas_call(
        _fused_kernel,
        grid_spec=pltpu.PrefetchScalarGridSpec(
            num_scalar_prefetch=1,
            grid=(QN // G,),
            in_specs=[
                pl.BlockSpec((BSZ, C_LEN, D), lambda k, t: (0, 0, 0)),
                pl.BlockSpec((1, G, QL, D), lambda k, t: (k, 0, 0, 0)),
                pl.BlockSpec((D, H), lambda k, t: (0, 0)),
                pl.BlockSpec((2 * D, H), lambda k, t: (0, 0)),
            ],
            out_specs=pl.BlockSpec((1, G, C_LEN, 2 * D), lambda k, t: (k, 0, 0, 0)),
            scratch_shapes=[
                pltpu.VMEM((BSZ, C_LEN, H), jnp.bfloat16),
                pltpu.VMEM((QL, D), jnp.float32),
                pltpu.SMEM((1,), jnp.int32),
            ],
        ),
        out_shape=jax.ShapeDtypeStruct((QN // G, G, C_LEN, 2 * D), jnp.float32),
        compiler_params=pltpu.CompilerParams(dimension_semantics=("arbitrary",)),
    )(tags32, ctx_b, q4, w1_b, w2_b)

    return out.reshape(QN, C_LEN, 2 * D)


# speedup vs baseline: 1.5531x; 1.3144x over previous
import math

import jax
import jax.numpy as jnp
from jax.experimental import pallas as pl
from jax.experimental.pallas import tpu as pltpu

BSZ = 8
C_LEN = 512
QN = 64
QL = 64
D = 512
H = 512
G = 4


def _question(cp, qp_b, cat):
    st = jax.lax.dot_general(
        qp_b, cp, (((1,), (1,)), ((), ())),
        preferred_element_type=jnp.float32,
    ) * (1.0 / math.sqrt(H))
    st = st - jnp.max(st, axis=0, keepdims=True)
    e = jnp.exp(st)
    attn_t = e / jnp.sum(e, axis=0, keepdims=True)
    attn = jnp.transpose(attn_t).astype(jnp.bfloat16)
    return jnp.dot(attn, cat, preferred_element_type=jnp.float32)


def _fused_kernel(tags_ref, ctx_ref, q_ref, w1_ref, w2_ref,
                  out_ref, cp_all, qsum_scr, pos_ref):
    k = pl.program_id(0)

    @pl.when(k == 0)
    def _():
        for b in range(BSZ):
            cp_all[b] = jax.nn.relu(
                jnp.dot(ctx_ref[b], w1_ref[...], preferred_element_type=jnp.float32)
            ).astype(jnp.bfloat16)

    p = pos_ref[0]
    S = qsum_scr[...]
    tprev = tags_ref[jnp.maximum(G * k - 1, 0)]
    tis = []
    cats = []
    for i in range(G):
        ti = tags_ref[G * k + i]
        seg = (ti != tprev) if i else ((k == 0) | (ti != tprev))
        p = jnp.where(seg, 0, p)
        S = jnp.where(seg, 0.0, S)
        inv = jnp.where(p == 0, 0.0, 1.0 / p.astype(jnp.float32))
        avg_b = (S * inv).astype(jnp.bfloat16)
        qf = q_ref[0, i]
        cats.append(jnp.concatenate(
            [qf.astype(jnp.bfloat16), avg_b], axis=1))
        tis.append(ti)
        S = S + qf
        p = p + 1
        tprev = ti
    qsum_scr[...] = S
    pos_ref[0] = p

    cat4 = jnp.concatenate(cats, axis=0)
    qp4 = jnp.dot(cat4, w2_ref[...], preferred_element_type=jnp.float32)
    qp4_b = jax.nn.relu(qp4).astype(jnp.bfloat16)

    for i in range(G):
        out_ref[0, i] = _question(cp_all[tis[i]],
                                  qp4_b[i * QL:(i + 1) * QL], cats[i])


def kernel(contexts, questions, tags, W1, W2):
    tags32 = tags.astype(jnp.int32)
    ctx_b = contexts.astype(jnp.bfloat16)
    w1_b = W1.astype(jnp.bfloat16)
    w2_b = W2.astype(jnp.bfloat16)
    q4 = questions.reshape(QN // G, G, QL, D)

    out = pl.pallas_call(
        _fused_kernel,
        grid_spec=pltpu.PrefetchScalarGridSpec(
            num_scalar_prefetch=1,
            grid=(QN // G,),
            in_specs=[
                pl.BlockSpec((BSZ, C_LEN, D), lambda k, t: (0, 0, 0)),
                pl.BlockSpec((1, G, QL, D), lambda k, t: (k, 0, 0, 0)),
                pl.BlockSpec((D, H), lambda k, t: (0, 0)),
                pl.BlockSpec((2 * D, H), lambda k, t: (0, 0)),
            ],
            out_specs=pl.BlockSpec((1, G, C_LEN, 2 * D), lambda k, t: (k, 0, 0, 0)),
            scratch_shapes=[
                pltpu.VMEM((BSZ, C_LEN, H), jnp.bfloat16),
                pltpu.VMEM((QL, D), jnp.float32),
                pltpu.SMEM((1,), jnp.int32),
            ],
        ),
        out_shape=jax.ShapeDtypeStruct((QN // G, G, C_LEN, 2 * D), jnp.float32),
        compiler_params=pltpu.CompilerParams(dimension_semantics=("arbitrary",)),
    )(tags32, ctx_b, q4, w1_b, w2_b)

    return out.reshape(QN, C_LEN, 2 * D)


# speedup vs baseline: 1.5792x; 1.0168x over previous
import math

import jax
import jax.numpy as jnp
from jax.experimental import pallas as pl
from jax.experimental.pallas import tpu as pltpu

BSZ = 8
C_LEN = 512
QN = 64
QL = 64
D = 512
H = 512
G = 8


def _question(cp, qp_b, cat):
    st = jax.lax.dot_general(
        qp_b, cp, (((1,), (1,)), ((), ())),
        preferred_element_type=jnp.float32,
    ) * (1.0 / math.sqrt(H))
    st = st - jnp.max(st, axis=0, keepdims=True)
    e = jnp.exp(st)
    attn_t = (e / jnp.sum(e, axis=0, keepdims=True)).astype(jnp.bfloat16)
    attn = jnp.transpose(attn_t)
    return jnp.dot(attn, cat, preferred_element_type=jnp.float32)


def _fused_kernel(tags_ref, ctx_ref, q_ref, w1_ref, w2_ref,
                  out_ref, cp_all, qsum_scr, pos_ref):
    k = pl.program_id(0)

    @pl.when(k == 0)
    def _():
        for b in range(BSZ):
            cp_all[b] = jax.nn.relu(
                jnp.dot(ctx_ref[b], w1_ref[...], preferred_element_type=jnp.float32)
            ).astype(jnp.bfloat16)

    p = pos_ref[0]
    S = qsum_scr[...]
    tprev = tags_ref[jnp.maximum(G * k - 1, 0)]
    tis = []
    cats = []
    for i in range(G):
        ti = tags_ref[G * k + i]
        seg = (ti != tprev) if i else ((k == 0) | (ti != tprev))
        p = jnp.where(seg, 0, p)
        S = jnp.where(seg, 0.0, S)
        inv = jnp.where(p == 0, 0.0, 1.0 / p.astype(jnp.float32))
        avg_b = (S * inv).astype(jnp.bfloat16)
        qf = q_ref[0, i]
        cats.append(jnp.concatenate(
            [qf.astype(jnp.bfloat16), avg_b], axis=1))
        tis.append(ti)
        S = S + qf
        p = p + 1
        tprev = ti
    qsum_scr[...] = S
    pos_ref[0] = p

    cat4 = jnp.concatenate(cats, axis=0)
    qp4 = jnp.dot(cat4, w2_ref[...], preferred_element_type=jnp.float32)
    qp4_b = jax.nn.relu(qp4).astype(jnp.bfloat16)

    for i in range(G):
        out_ref[0, i] = _question(cp_all[tis[i]],
                                  qp4_b[i * QL:(i + 1) * QL], cats[i])


def kernel(contexts, questions, tags, W1, W2):
    tags32 = tags.astype(jnp.int32)
    ctx_b = contexts.astype(jnp.bfloat16)
    w1_b = W1.astype(jnp.bfloat16)
    w2_b = W2.astype(jnp.bfloat16)
    q4 = questions.reshape(QN // G, G, QL, D)

    out = pl.pallas_call(
        _fused_kernel,
        grid_spec=pltpu.PrefetchScalarGridSpec(
            num_scalar_prefetch=1,
            grid=(QN // G,),
            in_specs=[
                pl.BlockSpec((BSZ, C_LEN, D), lambda k, t: (0, 0, 0)),
                pl.BlockSpec((1, G, QL, D), lambda k, t: (k, 0, 0, 0)),
                pl.BlockSpec((D, H), lambda k, t: (0, 0)),
                pl.BlockSpec((2 * D, H), lambda k, t: (0, 0)),
            ],
            out_specs=pl.BlockSpec((1, G, C_LEN, 2 * D), lambda k, t: (k, 0, 0, 0)),
            scratch_shapes=[
                pltpu.VMEM((BSZ, C_LEN, H), jnp.bfloat16),
                pltpu.VMEM((QL, D), jnp.float32),
                pltpu.SMEM((1,), jnp.int32),
            ],
        ),
        out_shape=jax.ShapeDtypeStruct((QN // G, G, C_LEN, 2 * D), jnp.float32),
        compiler_params=pltpu.CompilerParams(dimension_semantics=("arbitrary",)),
    )(tags32, ctx_b, q4, w1_b, w2_b)

    return out.reshape(QN, C_LEN, 2 * D)
